# baseline (speedup 1.0000x reference)
import functools

import jax
import jax.numpy as jnp
from jax.experimental import pallas as pl


def _add_pos_kernel(x_ref, pos_ref, o_ref):
    o_ref[...] = x_ref[...] + pos_ref[...]


@functools.partial(jax.jit, static_argnames=())
def kernel(x, pos_table):
    B, L, D = x.shape
    BB = 128
    grid = (B // BB,)
    return pl.pallas_call(
        _add_pos_kernel,
        grid=grid,
        in_specs=[
            pl.BlockSpec((BB, L, D), lambda i: (i, 0, 0)),
            pl.BlockSpec((L, D), lambda i: (0, 0)),
        ],
        out_specs=pl.BlockSpec((BB, L, D), lambda i: (i, 0, 0)),
        out_shape=jax.ShapeDtypeStruct((B, L, D), x.dtype),
    )(x, pos_table)


# speedup vs baseline: 1.6715x; 1.6715x over previous
import jax
import jax.numpy as jnp
from jax.experimental import pallas as pl


def _add_pos_kernel(x_ref, pos_ref, o_ref):
    o_ref[...] = x_ref[...] + pos_ref[...]


def kernel(x, pos_table):
    B, L, D = x.shape
    x2 = x.reshape(B, L * D)
    pos_row = jax.lax.slice(pos_table, (0, 0), (L, D)).reshape(1, L * D)
    BB = 256
    out2 = pl.pallas_call(
        _add_pos_kernel,
        grid=(B // BB,),
        in_specs=[
            pl.BlockSpec((BB, L * D), lambda i: (i, 0)),
            pl.BlockSpec((1, L * D), lambda i: (0, 0)),
        ],
        out_specs=pl.BlockSpec((BB, L * D), lambda i: (i, 0)),
        out_shape=jax.ShapeDtypeStruct((B, L * D), x.dtype),
    )(x2, pos_row)
    return out2.reshape(B, L, D)
